# Initial kernel scaffold; baseline (speedup 1.0000x reference)
import jax
import jax.numpy as jnp
from jax.experimental import pallas as pl


def kernel(h_p, h_path, edge_index, W, b):
    raise NotImplementedError("write your pallas kernel here")


# speedup vs baseline: 3.6276x; 3.6276x over previous
import functools

import jax
import jax.numpy as jnp
from jax import lax
from jax.experimental import pallas as pl
from jax.experimental.pallas import tpu as pltpu
from jax.experimental.pallas import tpu_sc as plsc

N_PROT = 10000
N_PATH = 10000
N_EDGE = 320000
D = 128

NC = 2
NS = 16
NW = NC * NS
CHUNK = 128
NCH = 80
NPHASE = 2
PH = NCH // NPHASE
E_PAD = NW * NCH * CHUNK
R_TILE = 640
R_PAD = NS * R_TILE


def _mm_body(h_ref, w_ref, o_ref):
    o_ref[...] = jnp.dot(h_ref[...], w_ref[...],
                         preferred_element_type=jnp.float32)


def _fin_body(p_ref, b_ref, o_ref):
    o_ref[...] = jnp.maximum(p_ref[0] + p_ref[1] + b_ref[...], 0.0)


def _sc_body(msg_hbm, src_hbm, dst_hbm, zeros_hbm, out_hbm,
             src_v, dst_v, buf0, buf1, agg, sem0, sem1):
    c = lax.axis_index("c")
    s = lax.axis_index("s")
    wid = s * NC + c

    pltpu.sync_copy(zeros_hbm, agg.at[pl.ds(s * R_TILE, R_TILE)])
    plsc.subcore_barrier()

    for p in range(NPHASE):
        row0 = (wid * NPHASE + p) * PH
        pltpu.sync_copy(src_hbm.at[pl.ds(row0, PH)], src_v)
        pltpu.sync_copy(dst_hbm.at[pl.ds(row0, PH)], dst_v)

        pltpu.async_copy(msg_hbm.at[src_v.at[0]], buf0, sem0)
        pltpu.async_copy(msg_hbm.at[src_v.at[1]], buf1, sem1)

        def body(k, _):
            j = 2 * k
            pltpu.make_async_copy(msg_hbm.at[src_v.at[j]], buf0, sem0).wait()
            pltpu.sync_copy(buf0, agg.at[dst_v.at[j]], add=True)
            pltpu.async_copy(msg_hbm.at[src_v.at[j + 2]], buf0, sem0)
            pltpu.make_async_copy(msg_hbm.at[src_v.at[j + 1]], buf1,
                                  sem1).wait()
            pltpu.sync_copy(buf1, agg.at[dst_v.at[j + 1]], add=True)
            pltpu.async_copy(msg_hbm.at[src_v.at[j + 3]], buf1, sem1)
            return 0

        lax.fori_loop(0, PH // 2 - 1, body, 0)

        pltpu.make_async_copy(msg_hbm.at[src_v.at[PH - 2]], buf0, sem0).wait()
        pltpu.sync_copy(buf0, agg.at[dst_v.at[PH - 2]], add=True)
        pltpu.make_async_copy(msg_hbm.at[src_v.at[PH - 1]], buf1, sem1).wait()
        pltpu.sync_copy(buf1, agg.at[dst_v.at[PH - 1]], add=True)

    plsc.subcore_barrier()
    pltpu.sync_copy(agg.at[pl.ds(s * R_TILE, R_TILE)],
                    out_hbm.at[pl.ds(c * R_PAD + s * R_TILE, R_TILE)])


_sc_scatter = functools.partial(
    pl.kernel,
    out_type=jax.ShapeDtypeStruct((NC * R_PAD, D), jnp.float32),
    mesh=plsc.VectorSubcoreMesh(core_axis_name="c", subcore_axis_name="s",
                                num_cores=NC, num_subcores=NS),
    scratch_types=[
        pltpu.VMEM((PH, CHUNK), jnp.int32),
        pltpu.VMEM((PH, CHUNK), jnp.int32),
        pltpu.VMEM((CHUNK, D), jnp.float32),
        pltpu.VMEM((CHUNK, D), jnp.float32),
        pltpu.VMEM_SHARED((R_PAD, D), jnp.float32),
        pltpu.SemaphoreType.DMA,
        pltpu.SemaphoreType.DMA,
    ],
)(_sc_body)


def kernel(h_p, h_path, edge_index, W, b):
    src = edge_index[0].astype(jnp.int32)
    dst = edge_index[1].astype(jnp.int32)
    pad = E_PAD - N_EDGE
    src = jnp.concatenate([src, jnp.zeros((pad,), jnp.int32)])
    dst = jnp.concatenate([dst, jnp.full((pad,), N_PROT, jnp.int32)])
    src3 = src.reshape(NW * NCH, CHUNK)
    dst3 = dst.reshape(NW * NCH, CHUNK)
    zeros = jnp.zeros((R_TILE, D), jnp.float32)

    msg = pl.pallas_call(
        _mm_body,
        grid=(10,),
        in_specs=[pl.BlockSpec((N_PATH // 10, D), lambda i: (i, 0)),
                  pl.BlockSpec((D, D), lambda i: (0, 0))],
        out_specs=pl.BlockSpec((N_PATH // 10, D), lambda i: (i, 0)),
        out_shape=jax.ShapeDtypeStruct((N_PATH, D), jnp.float32),
    )(h_path, W)

    partials = _sc_scatter(msg, src3, dst3, zeros)
    partials = partials.reshape(NC, R_PAD, D)

    out = pl.pallas_call(
        _fin_body,
        grid=(10,),
        in_specs=[pl.BlockSpec((NC, N_PROT // 10, D), lambda i: (0, i, 0)),
                  pl.BlockSpec((1, D), lambda i: (0, 0))],
        out_specs=pl.BlockSpec((N_PROT // 10, D), lambda i: (i, 0)),
        out_shape=jax.ShapeDtypeStruct((N_PROT, D), jnp.float32),
    )(partials, b.reshape(1, D))
    return out


# speedup vs baseline: 3.6331x; 1.0015x over previous
import functools

import jax
import jax.numpy as jnp
from jax import lax
from jax.experimental import pallas as pl
from jax.experimental.pallas import tpu as pltpu
from jax.experimental.pallas import tpu_sc as plsc

N_PROT = 10000
N_PATH = 10000
N_EDGE = 320000
D = 128

NC = 2
NS = 16
NW = NC * NS
CHUNK = 128
NCH = 80
NPHASE = 2
PH = NCH // NPHASE
E_PAD = NW * NCH * CHUNK
R_TILE = 640
R_PAD = NS * R_TILE


def _mm_body(h_ref, w_ref, o_ref):
    o_ref[...] = jnp.dot(h_ref[...], w_ref[...],
                         preferred_element_type=jnp.float32)


def _fin_body(p_ref, b_ref, o_ref):
    o_ref[...] = jnp.maximum(p_ref[0] + p_ref[1] + b_ref[...], 0.0)


def _sc_body(msg_hbm, src_hbm, dst_hbm, zeros_hbm, out_hbm,
             src_v, dst_v, buf0, buf1, agg, sem0, sem1):
    c = lax.axis_index("c")
    s = lax.axis_index("s")
    wid = s * NC + c

    pltpu.sync_copy(zeros_hbm, agg.at[pl.ds(s * R_TILE, R_TILE)])
    plsc.subcore_barrier()

    for p in range(NPHASE):
        row0 = (wid * NPHASE + p) * PH
        pltpu.sync_copy(src_hbm.at[pl.ds(row0, PH)], src_v)
        pltpu.sync_copy(dst_hbm.at[pl.ds(row0, PH)], dst_v)

        pltpu.async_copy(msg_hbm.at[src_v.at[0]], buf0, sem0)
        pltpu.async_copy(msg_hbm.at[src_v.at[1]], buf1, sem1)

        def body(k, _):
            j = 2 * k
            pltpu.make_async_copy(msg_hbm.at[src_v.at[j]], buf0, sem0).wait()
            pltpu.sync_copy(buf0, agg.at[dst_v.at[j]], add=True)
            pltpu.async_copy(msg_hbm.at[src_v.at[j + 2]], buf0, sem0)
            pltpu.make_async_copy(msg_hbm.at[src_v.at[j + 1]], buf1,
                                  sem1).wait()
            pltpu.sync_copy(buf1, agg.at[dst_v.at[j + 1]], add=True)
            pltpu.async_copy(msg_hbm.at[src_v.at[j + 3]], buf1, sem1)
            return 0

        lax.fori_loop(0, PH // 2 - 1, body, 0)

        pltpu.make_async_copy(msg_hbm.at[src_v.at[PH - 2]], buf0, sem0).wait()
        pltpu.sync_copy(buf0, agg.at[dst_v.at[PH - 2]], add=True)
        pltpu.make_async_copy(msg_hbm.at[src_v.at[PH - 1]], buf1, sem1).wait()
        pltpu.sync_copy(buf1, agg.at[dst_v.at[PH - 1]], add=True)

    plsc.subcore_barrier()
    pltpu.sync_copy(agg.at[pl.ds(s * R_TILE, R_TILE)],
                    out_hbm.at[pl.ds(c * R_PAD + s * R_TILE, R_TILE)])


_sc_scatter = functools.partial(
    pl.kernel,
    out_type=jax.ShapeDtypeStruct((NC * R_PAD, D), jnp.float32),
    mesh=plsc.VectorSubcoreMesh(core_axis_name="c", subcore_axis_name="s",
                                num_cores=NC, num_subcores=NS),
    scratch_types=[
        pltpu.VMEM((PH, CHUNK), jnp.int32),
        pltpu.VMEM((PH, CHUNK), jnp.int32),
        pltpu.VMEM((CHUNK, D), jnp.float32),
        pltpu.VMEM((CHUNK, D), jnp.float32),
        pltpu.VMEM_SHARED((R_PAD, D), jnp.float32),
        pltpu.SemaphoreType.DMA,
        pltpu.SemaphoreType.DMA,
    ],
)(_sc_body)


def kernel(h_p, h_path, edge_index, W, b):
    src = edge_index[0].astype(jnp.int32)
    dst = edge_index[1].astype(jnp.int32)
    pad = E_PAD - N_EDGE
    src = jnp.concatenate([src, jnp.zeros((pad,), jnp.int32)])
    pad_dst = N_PROT + (jnp.arange(pad, dtype=jnp.int32) % (R_PAD - N_PROT))
    dst = jnp.concatenate([dst, pad_dst])
    src3 = src.reshape(NW * NCH, CHUNK)
    dst3 = dst.reshape(NW * NCH, CHUNK)
    zeros = jnp.zeros((R_TILE, D), jnp.float32)

    msg = pl.pallas_call(
        _mm_body,
        grid=(10,),
        in_specs=[pl.BlockSpec((N_PATH // 10, D), lambda i: (i, 0)),
                  pl.BlockSpec((D, D), lambda i: (0, 0))],
        out_specs=pl.BlockSpec((N_PATH // 10, D), lambda i: (i, 0)),
        out_shape=jax.ShapeDtypeStruct((N_PATH, D), jnp.float32),
    )(h_path, W)

    partials = _sc_scatter(msg, src3, dst3, zeros)
    partials = partials.reshape(NC, R_PAD, D)

    out = pl.pallas_call(
        _fin_body,
        grid=(10,),
        in_specs=[pl.BlockSpec((NC, N_PROT // 10, D), lambda i: (0, i, 0)),
                  pl.BlockSpec((1, D), lambda i: (0, 0))],
        out_specs=pl.BlockSpec((N_PROT // 10, D), lambda i: (i, 0)),
        out_shape=jax.ShapeDtypeStruct((N_PROT, D), jnp.float32),
    )(partials, b.reshape(1, D))
    return out


# speedup vs baseline: 3.8712x; 1.0655x over previous
import functools

import jax
import jax.numpy as jnp
from jax import lax
from jax.experimental import pallas as pl
from jax.experimental.pallas import tpu as pltpu
from jax.experimental.pallas import tpu_sc as plsc

N_PROT = 10000
N_PATH = 10000
N_EDGE = 320000
D = 128

NC = 2
NS = 16
CHUNK = 128
PH = 32
NPH0 = 4
NPH1 = 1
NCHT = NS * PH * (NPH0 + NPH1)
E_PAD = NCHT * CHUNK
R_TILE = 640
R_PAD = NS * R_TILE


def _mm_body(h_ref, w_ref, o_ref):
    o_ref[...] = jnp.dot(h_ref[...], w_ref[...],
                         preferred_element_type=jnp.float32)


def _fin_body(p_ref, b_ref, o_ref):
    o_ref[...] = jnp.maximum(p_ref[0] + p_ref[1] + b_ref[...], 0.0)


def _sc_body(msg_hbm, src_hbm, dst_hbm, zeros_hbm, out_hbm,
             src_v, dst_v, buf0, buf1, agg, sem0, sem1):
    c = lax.axis_index("c")
    s = lax.axis_index("s")

    pltpu.sync_copy(zeros_hbm, agg.at[pl.ds(s * R_TILE, R_TILE)])
    plsc.subcore_barrier()

    n_phases = jnp.where(c == 0, NPH0, NPH1)

    def phase(p, _):
        row0 = jnp.where(c == 0, (s * NPH0 + p) * PH,
                         NS * NPH0 * PH + s * PH)
        pltpu.sync_copy(src_hbm.at[pl.ds(row0, PH)], src_v)
        pltpu.sync_copy(dst_hbm.at[pl.ds(row0, PH)], dst_v)

        pltpu.async_copy(msg_hbm.at[src_v.at[0]], buf0, sem0)
        pltpu.async_copy(msg_hbm.at[src_v.at[1]], buf1, sem1)

        def body(k, _):
            j = 2 * k
            pltpu.make_async_copy(msg_hbm.at[src_v.at[j]], buf0, sem0).wait()
            pltpu.sync_copy(buf0, agg.at[dst_v.at[j]], add=True)
            pltpu.async_copy(msg_hbm.at[src_v.at[j + 2]], buf0, sem0)
            pltpu.make_async_copy(msg_hbm.at[src_v.at[j + 1]], buf1,
                                  sem1).wait()
            pltpu.sync_copy(buf1, agg.at[dst_v.at[j + 1]], add=True)
            pltpu.async_copy(msg_hbm.at[src_v.at[j + 3]], buf1, sem1)
            return 0

        lax.fori_loop(0, PH // 2 - 1, body, 0)

        pltpu.make_async_copy(msg_hbm.at[src_v.at[PH - 2]], buf0, sem0).wait()
        pltpu.sync_copy(buf0, agg.at[dst_v.at[PH - 2]], add=True)
        pltpu.make_async_copy(msg_hbm.at[src_v.at[PH - 1]], buf1, sem1).wait()
        pltpu.sync_copy(buf1, agg.at[dst_v.at[PH - 1]], add=True)
        return 0

    lax.fori_loop(0, n_phases, phase, 0)

    plsc.subcore_barrier()
    pltpu.sync_copy(agg.at[pl.ds(s * R_TILE, R_TILE)],
                    out_hbm.at[pl.ds(c * R_PAD + s * R_TILE, R_TILE)])


_sc_scatter = functools.partial(
    pl.kernel,
    out_type=jax.ShapeDtypeStruct((NC * R_PAD, D), jnp.float32),
    mesh=plsc.VectorSubcoreMesh(core_axis_name="c", subcore_axis_name="s",
                                num_cores=NC, num_subcores=NS),
    scratch_types=[
        pltpu.VMEM((PH, CHUNK), jnp.int32),
        pltpu.VMEM((PH, CHUNK), jnp.int32),
        pltpu.VMEM((CHUNK, D), jnp.float32),
        pltpu.VMEM((CHUNK, D), jnp.float32),
        pltpu.VMEM_SHARED((R_PAD, D), jnp.float32),
        pltpu.SemaphoreType.DMA,
        pltpu.SemaphoreType.DMA,
    ],
)(_sc_body)


def kernel(h_p, h_path, edge_index, W, b):
    src = edge_index[0].astype(jnp.int32)
    dst = edge_index[1].astype(jnp.int32)
    pad = E_PAD - N_EDGE
    src = jnp.concatenate([src, jnp.zeros((pad,), jnp.int32)])
    pad_dst = N_PROT + (jnp.arange(pad, dtype=jnp.int32) % (R_PAD - N_PROT))
    dst = jnp.concatenate([dst, pad_dst])
    src3 = src.reshape(NCHT, CHUNK)
    dst3 = dst.reshape(NCHT, CHUNK)
    zeros = jnp.zeros((R_TILE, D), jnp.float32)

    msg = pl.pallas_call(
        _mm_body,
        grid=(10,),
        in_specs=[pl.BlockSpec((N_PATH // 10, D), lambda i: (i, 0)),
                  pl.BlockSpec((D, D), lambda i: (0, 0))],
        out_specs=pl.BlockSpec((N_PATH // 10, D), lambda i: (i, 0)),
        out_shape=jax.ShapeDtypeStruct((N_PATH, D), jnp.float32),
    )(h_path, W)

    partials = _sc_scatter(msg, src3, dst3, zeros)
    partials = partials.reshape(NC, R_PAD, D)

    out = pl.pallas_call(
        _fin_body,
        grid=(10,),
        in_specs=[pl.BlockSpec((NC, N_PROT // 10, D), lambda i: (0, i, 0)),
                  pl.BlockSpec((1, D), lambda i: (0, 0))],
        out_specs=pl.BlockSpec((N_PROT // 10, D), lambda i: (i, 0)),
        out_shape=jax.ShapeDtypeStruct((N_PROT, D), jnp.float32),
    )(partials, b.reshape(1, D))
    return out
